# baseline (speedup 1.0000x reference)
import functools

import jax
import jax.numpy as jnp
from jax.experimental import pallas as pl
from jax.experimental.pallas import tpu as pltpu
from jax.experimental.pallas import tpu_sc as plsc

_K = 8192
_C = 32
_TILE_M = 256
_CK = 512
_NCHUNK = _K // _CK

_NCORES = 2
_NSUB = 16
_NW = _NCORES * _NSUB


def _vq_argmin_kernel(xt_ref, emb_ref, cs_ref, idx_ref, loss_ref, used_ref,
                      minval_scr, amin_scr, s_scr):
    m = pl.program_id(0)
    k = pl.program_id(1)

    xt = xt_ref[...]
    iota = jax.lax.broadcasted_iota(jnp.int32, (_CK, _TILE_M), 0)

    @pl.when(jnp.logical_and(m == 0, k == 0))
    def _global_init():
        loss_ref[...] = jnp.zeros_like(loss_ref)
        used_ref[...] = (jnp.sum((cs_ref[...] > 1e-05).astype(jnp.float32))
                         / float(_K)).reshape(1, 1)

    kp = jax.lax.rem(k, _NCHUNK)
    embc = emb_ref[pl.ds(kp * _CK, _CK), :]
    e2h = 0.5 * jnp.sum(embc * embc, axis=1, keepdims=True)
    xe = jax.lax.dot_general(embc, xt, (((1,), (0,)), ((), ())),
                             preferred_element_type=jnp.float32)
    s_scr[jax.lax.rem(k, 2)] = e2h - xe

    kk = k - 1
    s = s_scr[jax.lax.rem(k + 1, 2)]
    mv = jnp.min(s, axis=0, keepdims=True)
    loc = jnp.min(jnp.where(s == mv, iota, _CK), axis=0,
                  keepdims=True) + kk * _CK
    fresh = k == 0
    upd = jnp.logical_and(jnp.logical_not(fresh), mv < minval_scr[...])
    inf_row = jnp.full((1, _TILE_M), jnp.inf, jnp.float32)
    minval_scr[...] = jnp.where(
        fresh, inf_row, jnp.where(upd, mv, minval_scr[...]))
    amin_scr[...] = jnp.where(
        fresh, jnp.zeros((1, _TILE_M), jnp.int32),
        jnp.where(upd, loc, amin_scr[...]))

    @pl.when(k == _NCHUNK)
    def _tile_done():
        idx_ref[0, 0, :] = amin_scr[0, :]
        x2 = jnp.sum(xt * xt, axis=0, keepdims=True)
        d2min = jnp.maximum(x2 + 2.0 * minval_scr[...], 0.0)
        loss_ref[...] += jnp.sum(d2min).reshape(1, 1)

        @pl.when(m == pl.num_programs(0) - 1)
        def _finalize():
            n_tok = jnp.float32(pl.num_programs(0)) * jnp.float32(_TILE_M)
            loss_ref[...] = loss_ref[...] / (n_tok * jnp.float32(_C))


def _tc_argmin(flat_xt, embedding, cs2d, n_tokens):
    n_m = n_tokens // _TILE_M
    return pl.pallas_call(
        _vq_argmin_kernel,
        grid=(n_m, _NCHUNK + 1),
        in_specs=[
            pl.BlockSpec((_C, _TILE_M), lambda m, k: (0, m)),
            pl.BlockSpec((_K, _C), lambda m, k: (0, 0)),
            pl.BlockSpec((1, _K), lambda m, k: (0, 0)),
        ],
        out_specs=[
            pl.BlockSpec((1, 1, _TILE_M), lambda m, k: (m, 0, 0)),
            pl.BlockSpec((1, 1), lambda m, k: (0, 0)),
            pl.BlockSpec((1, 1), lambda m, k: (0, 0)),
        ],
        out_shape=[
            jax.ShapeDtypeStruct((n_m, 1, _TILE_M), jnp.int32),
            jax.ShapeDtypeStruct((1, 1), jnp.float32),
            jax.ShapeDtypeStruct((1, 1), jnp.float32),
        ],
        scratch_shapes=[
            pltpu.VMEM((1, _TILE_M), jnp.float32),
            pltpu.VMEM((1, _TILE_M), jnp.int32),
            pltpu.VMEM((2, _CK, _TILE_M), jnp.float32),
        ],
    )(flat_xt, embedding, cs2d)


_RPW = (_K // _NW) // 128


def _vq_sc_kernel(emb_hbm, idx_hbm, zeros_hbm, ones_hbm, q_hbm, counts_hbm,
                  idx_v, rows_v, ones_v, cshared, sem):
    c = jax.lax.axis_index("c")
    s = jax.lax.axis_index("s")
    wid = c * _NSUB + s

    pltpu.sync_copy(idx_hbm.at[pl.ds(wid * _RPW, _RPW)], idx_v)
    copies = [
        pltpu.async_copy(emb_hbm.at[idx_v.at[j]], rows_v.at[j], sem)
        for j in range(_RPW)
    ]

    @pl.when(s == 0)
    def _zero_counts():
        pltpu.sync_copy(zeros_hbm, cshared)

    pltpu.sync_copy(ones_hbm, ones_v)
    for copy in copies:
        copy.wait()
    pltpu.sync_copy(rows_v, q_hbm.at[pl.ds(wid * _RPW, _RPW)])

    plsc.subcore_barrier()
    for j in range(_RPW):
        pltpu.sync_copy(ones_v, cshared.at[idx_v.at[j]], add=True)
    plsc.subcore_barrier()

    @pl.when(s == 0)
    def _dump_counts():
        pltpu.sync_copy(cshared, counts_hbm.at[c])


@functools.partial(
    pl.kernel,
    mesh=plsc.VectorSubcoreMesh(core_axis_name="c", subcore_axis_name="s"),
    out_type=[
        jax.ShapeDtypeStruct((_K // 128, 128, 128), jnp.float32),
        jax.ShapeDtypeStruct((_NCORES, _K), jnp.float32),
    ],
    scratch_types=[
        pltpu.VMEM((_RPW, 128), jnp.int32),
        pltpu.VMEM((_RPW, 128, 128), jnp.float32),
        pltpu.VMEM((128,), jnp.float32),
        pltpu.VMEM_SHARED((_K,), jnp.float32),
        pltpu.SemaphoreType.DMA,
    ],
)
def _sc_gather_hist(emb_hbm, idx_hbm, zeros_hbm, ones_hbm, q_hbm, counts_hbm,
                    idx_v, rows_v, ones_v, cshared, sem):
    _vq_sc_kernel(emb_hbm, idx_hbm, zeros_hbm, ones_hbm, q_hbm, counts_hbm,
                  idx_v, rows_v, ones_v, cshared, sem)


def _perp_kernel(counts_ref, perp_ref):
    counts = counts_ref[0, :] + counts_ref[1, :]
    avg = counts / jnp.float32(_K)
    perp_ref[...] = jnp.exp(
        -jnp.sum(avg * jnp.log(avg + 1e-10))).reshape(1, 1)


def _tc_perplexity(counts2):
    return pl.pallas_call(
        _perp_kernel,
        out_shape=jax.ShapeDtypeStruct((1, 1), jnp.float32),
    )(counts2)


def kernel(z, embedding, cluster_size):
    B, C, D, H, W = z.shape
    K = embedding.shape[0]
    n_tokens = B * D * H * W

    flat_xt = jnp.transpose(z, (1, 0, 2, 3, 4)).reshape(C, n_tokens)
    cs2d = cluster_size.reshape(1, K)

    idx3, loss, used = _tc_argmin(flat_xt, embedding, cs2d, n_tokens)
    idx_rows = idx3.reshape(n_tokens // 128, 128)

    emb_pad = jnp.pad(embedding, ((0, 0), (0, 128 - C)))
    zeros = jnp.zeros((K,), jnp.float32)
    ones = jnp.ones((128,), jnp.float32)
    qpad, counts2 = _sc_gather_hist(emb_pad, idx_rows, zeros, ones)

    perp = _tc_perplexity(counts2)

    q = qpad.reshape(n_tokens, 128)[:, :C]
    quantized = q.reshape(B, D, H, W, C).transpose(0, 4, 1, 2, 3)
    encoding_indices = idx3.reshape(B, D, H, W)
    return (quantized, loss[0, 0], encoding_indices, perp[0, 0], used[0, 0])


# speedup vs baseline: 1.4097x; 1.4097x over previous
import functools

import jax
import jax.numpy as jnp
from jax.experimental import pallas as pl
from jax.experimental.pallas import tpu as pltpu
from jax.experimental.pallas import tpu_sc as plsc

_K = 8192
_C = 32
_TILE_M = 256
_CK = 1024
_NCHUNK = _K // _CK

_NCORES = 2
_NSUB = 16
_NW = _NCORES * _NSUB


def _vq_argmin_kernel(xt_ref, emb_ref, cs_ref, idx_ref, loss_ref, used_ref,
                      minval_scr, amin_scr):
    m = pl.program_id(0)
    k = pl.program_id(1)

    xt = xt_ref[...]
    iota = jax.lax.broadcasted_iota(jnp.int32, (_CK, _TILE_M), 0)

    @pl.when(jnp.logical_and(m == 0, k == 0))
    def _global_init():
        loss_ref[...] = jnp.zeros_like(loss_ref)
        used_ref[...] = (jnp.sum((cs_ref[...] > 1e-05).astype(jnp.float32))
                         / float(_K)).reshape(1, 1)

    embc = emb_ref[pl.ds(k * _CK, _CK), :]
    e2h = 0.5 * jnp.sum(embc * embc, axis=1, keepdims=True)
    xe = jax.lax.dot_general(embc, xt, (((1,), (0,)), ((), ())),
                             preferred_element_type=jnp.float32)
    s = e2h - xe
    mv = jnp.min(s, axis=0, keepdims=True)
    loc = jnp.min(jnp.where(s == mv, iota, _CK), axis=0,
                  keepdims=True) + k * _CK
    fresh = k == 0
    upd = jnp.logical_and(jnp.logical_not(fresh), mv < minval_scr[...])
    inf_row = jnp.full((1, _TILE_M), jnp.inf, jnp.float32)
    minval_scr[...] = jnp.where(
        fresh, jnp.minimum(mv, inf_row), jnp.where(upd, mv, minval_scr[...]))
    amin_scr[...] = jnp.where(
        fresh, loc, jnp.where(upd, loc, amin_scr[...]))

    @pl.when(k == _NCHUNK - 1)
    def _tile_done():
        idx_ref[0, 0, :] = amin_scr[0, :]
        x2 = jnp.sum(xt * xt, axis=0, keepdims=True)
        d2min = jnp.maximum(x2 + 2.0 * minval_scr[...], 0.0)
        loss_ref[...] += jnp.sum(d2min).reshape(1, 1)

        @pl.when(m == pl.num_programs(0) - 1)
        def _finalize():
            n_tok = jnp.float32(pl.num_programs(0)) * jnp.float32(_TILE_M)
            loss_ref[...] = loss_ref[...] / (n_tok * jnp.float32(_C))


def _tc_argmin(flat_xt, embedding, cs2d, n_tokens):
    n_m = n_tokens // _TILE_M
    return pl.pallas_call(
        _vq_argmin_kernel,
        grid=(n_m, _NCHUNK),
        in_specs=[
            pl.BlockSpec((_C, _TILE_M), lambda m, k: (0, m)),
            pl.BlockSpec((_K, _C), lambda m, k: (0, 0)),
            pl.BlockSpec((1, _K), lambda m, k: (0, 0)),
        ],
        out_specs=[
            pl.BlockSpec((1, 1, _TILE_M), lambda m, k: (m, 0, 0)),
            pl.BlockSpec((1, 1), lambda m, k: (0, 0)),
            pl.BlockSpec((1, 1), lambda m, k: (0, 0)),
        ],
        out_shape=[
            jax.ShapeDtypeStruct((n_m, 1, _TILE_M), jnp.int32),
            jax.ShapeDtypeStruct((1, 1), jnp.float32),
            jax.ShapeDtypeStruct((1, 1), jnp.float32),
        ],
        scratch_shapes=[
            pltpu.VMEM((1, _TILE_M), jnp.float32),
            pltpu.VMEM((1, _TILE_M), jnp.int32),
        ],
    )(flat_xt, embedding, cs2d)


_RPW = (_K // _NW) // 128


def _vq_sc_kernel(emb_hbm, idx_hbm, zeros_hbm, ones_hbm, q_hbm, counts_hbm,
                  idx_v, rows_v, ones_v, cshared, sem):
    c = jax.lax.axis_index("c")
    s = jax.lax.axis_index("s")
    wid = c * _NSUB + s

    pltpu.sync_copy(idx_hbm.at[pl.ds(wid * _RPW, _RPW)], idx_v)
    copies = [
        pltpu.async_copy(emb_hbm.at[idx_v.at[j]], rows_v.at[j], sem)
        for j in range(_RPW)
    ]

    @pl.when(s == 0)
    def _zero_counts():
        pltpu.sync_copy(zeros_hbm, cshared)

    pltpu.sync_copy(ones_hbm, ones_v)
    for copy in copies:
        copy.wait()
    pltpu.sync_copy(rows_v, q_hbm.at[pl.ds(wid * _RPW, _RPW)])

    plsc.subcore_barrier()
    for j in range(_RPW):
        pltpu.sync_copy(ones_v, cshared.at[idx_v.at[j]], add=True)
    plsc.subcore_barrier()

    @pl.when(s == 0)
    def _dump_counts():
        pltpu.sync_copy(cshared, counts_hbm.at[c])


@functools.partial(
    pl.kernel,
    mesh=plsc.VectorSubcoreMesh(core_axis_name="c", subcore_axis_name="s"),
    out_type=[
        jax.ShapeDtypeStruct((_K // 128, 128, 128), jnp.float32),
        jax.ShapeDtypeStruct((_NCORES, _K), jnp.float32),
    ],
    scratch_types=[
        pltpu.VMEM((_RPW, 128), jnp.int32),
        pltpu.VMEM((_RPW, 128, 128), jnp.float32),
        pltpu.VMEM((128,), jnp.float32),
        pltpu.VMEM_SHARED((_K,), jnp.float32),
        pltpu.SemaphoreType.DMA,
    ],
)
def _sc_gather_hist(emb_hbm, idx_hbm, zeros_hbm, ones_hbm, q_hbm, counts_hbm,
                    idx_v, rows_v, ones_v, cshared, sem):
    _vq_sc_kernel(emb_hbm, idx_hbm, zeros_hbm, ones_hbm, q_hbm, counts_hbm,
                  idx_v, rows_v, ones_v, cshared, sem)


def _perp_kernel(counts_ref, perp_ref):
    counts = counts_ref[0, :] + counts_ref[1, :]
    avg = counts / jnp.float32(_K)
    perp_ref[...] = jnp.exp(
        -jnp.sum(avg * jnp.log(avg + 1e-10))).reshape(1, 1)


def _tc_perplexity(counts2):
    return pl.pallas_call(
        _perp_kernel,
        out_shape=jax.ShapeDtypeStruct((1, 1), jnp.float32),
    )(counts2)


def kernel(z, embedding, cluster_size):
    B, C, D, H, W = z.shape
    K = embedding.shape[0]
    n_tokens = B * D * H * W

    flat_xt = jnp.transpose(z, (1, 0, 2, 3, 4)).reshape(C, n_tokens)
    cs2d = cluster_size.reshape(1, K)

    idx3, loss, used = _tc_argmin(flat_xt, embedding, cs2d, n_tokens)
    idx_rows = idx3.reshape(n_tokens // 128, 128)

    emb_pad = jnp.pad(embedding, ((0, 0), (0, 128 - C)))
    zeros = jnp.zeros((K,), jnp.float32)
    ones = jnp.ones((128,), jnp.float32)
    qpad, counts2 = _sc_gather_hist(emb_pad, idx_rows, zeros, ones)

    perp = _tc_perplexity(counts2)

    q = qpad.reshape(n_tokens, 128)[:, :C]
    quantized = q.reshape(B, D, H, W, C).transpose(0, 4, 1, 2, 3)
    encoding_indices = idx3.reshape(B, D, H, W)
    return (quantized, loss[0, 0], encoding_indices, perp[0, 0], used[0, 0])


# speedup vs baseline: 2.1598x; 1.5321x over previous
import functools

import jax
import jax.numpy as jnp
from jax.experimental import pallas as pl
from jax.experimental.pallas import tpu as pltpu
from jax.experimental.pallas import tpu_sc as plsc

_K = 8192
_C = 32
_TILE_M = 1024
_CK = 8192
_NCHUNK = _K // _CK

_NCORES = 2
_NSUB = 16
_NW = _NCORES * _NSUB


def _vq_argmin_kernel(xt_ref, emb_ref, cs_ref, idx_ref, loss_ref, used_ref,
                      minval_scr, amin_scr):
    m = pl.program_id(0)
    k = pl.program_id(1)

    xt = xt_ref[...]
    iota = jax.lax.broadcasted_iota(jnp.int32, (_CK, _TILE_M), 0)

    @pl.when(jnp.logical_and(m == 0, k == 0))
    def _global_init():
        loss_ref[...] = jnp.zeros_like(loss_ref)
        used_ref[...] = (jnp.sum((cs_ref[...] > 1e-05).astype(jnp.float32))
                         / float(_K)).reshape(1, 1)

    embc = emb_ref[pl.ds(k * _CK, _CK), :]
    e2h = 0.5 * jnp.sum(embc * embc, axis=1, keepdims=True)
    xe = jax.lax.dot_general(embc, xt, (((1,), (0,)), ((), ())),
                             preferred_element_type=jnp.float32)
    s = e2h - xe
    mv = jnp.min(s, axis=0, keepdims=True)
    loc = jnp.min(jnp.where(s == mv, iota, _CK), axis=0,
                  keepdims=True) + k * _CK
    fresh = k == 0
    upd = jnp.logical_and(jnp.logical_not(fresh), mv < minval_scr[...])
    inf_row = jnp.full((1, _TILE_M), jnp.inf, jnp.float32)
    minval_scr[...] = jnp.where(
        fresh, jnp.minimum(mv, inf_row), jnp.where(upd, mv, minval_scr[...]))
    amin_scr[...] = jnp.where(
        fresh, loc, jnp.where(upd, loc, amin_scr[...]))

    @pl.when(k == _NCHUNK - 1)
    def _tile_done():
        idx_ref[0, 0, :] = amin_scr[0, :]
        x2 = jnp.sum(xt * xt, axis=0, keepdims=True)
        d2min = jnp.maximum(x2 + 2.0 * minval_scr[...], 0.0)
        loss_ref[...] += jnp.sum(d2min).reshape(1, 1)

        @pl.when(m == pl.num_programs(0) - 1)
        def _finalize():
            n_tok = jnp.float32(pl.num_programs(0)) * jnp.float32(_TILE_M)
            loss_ref[...] = loss_ref[...] / (n_tok * jnp.float32(_C))


def _tc_argmin(flat_xt, embedding, cs2d, n_tokens):
    n_m = n_tokens // _TILE_M
    return pl.pallas_call(
        _vq_argmin_kernel,
        grid=(n_m, _NCHUNK),
        in_specs=[
            pl.BlockSpec((_C, _TILE_M), lambda m, k: (0, m)),
            pl.BlockSpec((_K, _C), lambda m, k: (0, 0)),
            pl.BlockSpec((1, _K), lambda m, k: (0, 0)),
        ],
        out_specs=[
            pl.BlockSpec((1, 1, _TILE_M), lambda m, k: (m, 0, 0)),
            pl.BlockSpec((1, 1), lambda m, k: (0, 0)),
            pl.BlockSpec((1, 1), lambda m, k: (0, 0)),
        ],
        out_shape=[
            jax.ShapeDtypeStruct((n_m, 1, _TILE_M), jnp.int32),
            jax.ShapeDtypeStruct((1, 1), jnp.float32),
            jax.ShapeDtypeStruct((1, 1), jnp.float32),
        ],
        scratch_shapes=[
            pltpu.VMEM((1, _TILE_M), jnp.float32),
            pltpu.VMEM((1, _TILE_M), jnp.int32),
        ],
    )(flat_xt, embedding, cs2d)


_RPW = (_K // _NW) // 128


def _vq_sc_kernel(emb_hbm, idx_hbm, zeros_hbm, ones_hbm, q_hbm, counts_hbm,
                  idx_v, rows_v, ones_v, cshared, sem):
    c = jax.lax.axis_index("c")
    s = jax.lax.axis_index("s")
    wid = c * _NSUB + s

    pltpu.sync_copy(idx_hbm.at[pl.ds(wid * _RPW, _RPW)], idx_v)
    copies = [
        pltpu.async_copy(emb_hbm.at[idx_v.at[j]], rows_v.at[j], sem)
        for j in range(_RPW)
    ]

    @pl.when(s == 0)
    def _zero_counts():
        pltpu.sync_copy(zeros_hbm, cshared)

    pltpu.sync_copy(ones_hbm, ones_v)
    for copy in copies:
        copy.wait()
    pltpu.sync_copy(rows_v, q_hbm.at[pl.ds(wid * _RPW, _RPW)])

    plsc.subcore_barrier()
    for j in range(_RPW):
        pltpu.sync_copy(ones_v, cshared.at[idx_v.at[j]], add=True)
    plsc.subcore_barrier()

    @pl.when(s == 0)
    def _dump_counts():
        pltpu.sync_copy(cshared, counts_hbm.at[c])


@functools.partial(
    pl.kernel,
    mesh=plsc.VectorSubcoreMesh(core_axis_name="c", subcore_axis_name="s"),
    out_type=[
        jax.ShapeDtypeStruct((_K // 128, 128, 128), jnp.float32),
        jax.ShapeDtypeStruct((_NCORES, _K), jnp.float32),
    ],
    scratch_types=[
        pltpu.VMEM((_RPW, 128), jnp.int32),
        pltpu.VMEM((_RPW, 128, 128), jnp.float32),
        pltpu.VMEM((128,), jnp.float32),
        pltpu.VMEM_SHARED((_K,), jnp.float32),
        pltpu.SemaphoreType.DMA,
    ],
)
def _sc_gather_hist(emb_hbm, idx_hbm, zeros_hbm, ones_hbm, q_hbm, counts_hbm,
                    idx_v, rows_v, ones_v, cshared, sem):
    _vq_sc_kernel(emb_hbm, idx_hbm, zeros_hbm, ones_hbm, q_hbm, counts_hbm,
                  idx_v, rows_v, ones_v, cshared, sem)


def _perp_kernel(counts_ref, perp_ref):
    counts = counts_ref[0, :] + counts_ref[1, :]
    avg = counts / jnp.float32(_K)
    perp_ref[...] = jnp.exp(
        -jnp.sum(avg * jnp.log(avg + 1e-10))).reshape(1, 1)


def _tc_perplexity(counts2):
    return pl.pallas_call(
        _perp_kernel,
        out_shape=jax.ShapeDtypeStruct((1, 1), jnp.float32),
    )(counts2)


def kernel(z, embedding, cluster_size):
    B, C, D, H, W = z.shape
    K = embedding.shape[0]
    n_tokens = B * D * H * W

    flat_xt = jnp.transpose(z, (1, 0, 2, 3, 4)).reshape(C, n_tokens)
    cs2d = cluster_size.reshape(1, K)

    idx3, loss, used = _tc_argmin(flat_xt, embedding, cs2d, n_tokens)
    idx_rows = idx3.reshape(n_tokens // 128, 128)

    emb_pad = jnp.pad(embedding, ((0, 0), (0, 128 - C)))
    zeros = jnp.zeros((K,), jnp.float32)
    ones = jnp.ones((128,), jnp.float32)
    qpad, counts2 = _sc_gather_hist(emb_pad, idx_rows, zeros, ones)

    perp = _tc_perplexity(counts2)

    q = qpad.reshape(n_tokens, 128)[:, :C]
    quantized = q.reshape(B, D, H, W, C).transpose(0, 4, 1, 2, 3)
    encoding_indices = idx3.reshape(B, D, H, W)
    return (quantized, loss[0, 0], encoding_indices, perp[0, 0], used[0, 0])


# speedup vs baseline: 2.2862x; 1.0585x over previous
import functools

import jax
import jax.numpy as jnp
from jax.experimental import pallas as pl
from jax.experimental.pallas import tpu as pltpu
from jax.experimental.pallas import tpu_sc as plsc

_K = 8192
_C = 32
_TILE_M = 2048
_CK = 8192
_NCHUNK = _K // _CK

_NCORES = 2
_NSUB = 16
_NW = _NCORES * _NSUB


def _vq_argmin_kernel(xt_ref, emb_ref, cs_ref, idx_ref, loss_ref, used_ref,
                      minval_scr, amin_scr):
    m = pl.program_id(0)
    k = pl.program_id(1)

    xt = xt_ref[...]
    iota = jax.lax.broadcasted_iota(jnp.int32, (_CK, _TILE_M), 0)

    @pl.when(jnp.logical_and(m == 0, k == 0))
    def _global_init():
        loss_ref[...] = jnp.zeros_like(loss_ref)
        used_ref[...] = (jnp.sum((cs_ref[...] > 1e-05).astype(jnp.float32))
                         / float(_K)).reshape(1, 1)

    embc = emb_ref[pl.ds(k * _CK, _CK), :]
    e2h = 0.5 * jnp.sum(embc * embc, axis=1, keepdims=True)
    xe = jax.lax.dot_general(embc, xt, (((1,), (0,)), ((), ())),
                             preferred_element_type=jnp.float32)
    s = e2h - xe
    mv = jnp.min(s, axis=0, keepdims=True)
    loc = jnp.min(jnp.where(s == mv, iota, _CK), axis=0,
                  keepdims=True) + k * _CK
    fresh = k == 0
    upd = jnp.logical_and(jnp.logical_not(fresh), mv < minval_scr[...])
    inf_row = jnp.full((1, _TILE_M), jnp.inf, jnp.float32)
    minval_scr[...] = jnp.where(
        fresh, jnp.minimum(mv, inf_row), jnp.where(upd, mv, minval_scr[...]))
    amin_scr[...] = jnp.where(
        fresh, loc, jnp.where(upd, loc, amin_scr[...]))

    @pl.when(k == _NCHUNK - 1)
    def _tile_done():
        idx_ref[0, 0, :] = amin_scr[0, :]
        x2 = jnp.sum(xt * xt, axis=0, keepdims=True)
        d2min = jnp.maximum(x2 + 2.0 * minval_scr[...], 0.0)
        loss_ref[...] += jnp.sum(d2min).reshape(1, 1)

        @pl.when(m == pl.num_programs(0) - 1)
        def _finalize():
            n_tok = jnp.float32(pl.num_programs(0)) * jnp.float32(_TILE_M)
            loss_ref[...] = loss_ref[...] / (n_tok * jnp.float32(_C))


def _tc_argmin(flat_xt, embedding, cs2d, n_tokens):
    n_m = n_tokens // _TILE_M
    return pl.pallas_call(
        _vq_argmin_kernel,
        grid=(n_m, _NCHUNK),
        in_specs=[
            pl.BlockSpec((_C, _TILE_M), lambda m, k: (0, m)),
            pl.BlockSpec((_K, _C), lambda m, k: (0, 0)),
            pl.BlockSpec((1, _K), lambda m, k: (0, 0)),
        ],
        out_specs=[
            pl.BlockSpec((1, 1, _TILE_M), lambda m, k: (m, 0, 0)),
            pl.BlockSpec((1, 1), lambda m, k: (0, 0)),
            pl.BlockSpec((1, 1), lambda m, k: (0, 0)),
        ],
        out_shape=[
            jax.ShapeDtypeStruct((n_m, 1, _TILE_M), jnp.int32),
            jax.ShapeDtypeStruct((1, 1), jnp.float32),
            jax.ShapeDtypeStruct((1, 1), jnp.float32),
        ],
        scratch_shapes=[
            pltpu.VMEM((1, _TILE_M), jnp.float32),
            pltpu.VMEM((1, _TILE_M), jnp.int32),
        ],
    )(flat_xt, embedding, cs2d)


_RPW = (_K // _NW) // 128


def _vq_sc_kernel(emb_hbm, idx_hbm, zeros_hbm, ones_hbm, q_hbm, counts_hbm,
                  idx_v, rows_v, ones_v, cshared, sem):
    c = jax.lax.axis_index("c")
    s = jax.lax.axis_index("s")
    wid = c * _NSUB + s

    pltpu.sync_copy(idx_hbm.at[pl.ds(wid * _RPW, _RPW)], idx_v)
    copies = [
        pltpu.async_copy(emb_hbm.at[idx_v.at[j]], rows_v.at[j], sem)
        for j in range(_RPW)
    ]

    @pl.when(s == 0)
    def _zero_counts():
        pltpu.sync_copy(zeros_hbm, cshared)

    pltpu.sync_copy(ones_hbm, ones_v)
    for copy in copies:
        copy.wait()
    pltpu.sync_copy(rows_v, q_hbm.at[pl.ds(wid * _RPW, _RPW)])

    plsc.subcore_barrier()
    for j in range(_RPW):
        pltpu.sync_copy(ones_v, cshared.at[idx_v.at[j]], add=True)
    plsc.subcore_barrier()

    @pl.when(s == 0)
    def _dump_counts():
        pltpu.sync_copy(cshared, counts_hbm.at[c])


@functools.partial(
    pl.kernel,
    mesh=plsc.VectorSubcoreMesh(core_axis_name="c", subcore_axis_name="s"),
    out_type=[
        jax.ShapeDtypeStruct((_K // 128, 128, 128), jnp.float32),
        jax.ShapeDtypeStruct((_NCORES, _K), jnp.float32),
    ],
    scratch_types=[
        pltpu.VMEM((_RPW, 128), jnp.int32),
        pltpu.VMEM((_RPW, 128, 128), jnp.float32),
        pltpu.VMEM((128,), jnp.float32),
        pltpu.VMEM_SHARED((_K,), jnp.float32),
        pltpu.SemaphoreType.DMA,
    ],
)
def _sc_gather_hist(emb_hbm, idx_hbm, zeros_hbm, ones_hbm, q_hbm, counts_hbm,
                    idx_v, rows_v, ones_v, cshared, sem):
    _vq_sc_kernel(emb_hbm, idx_hbm, zeros_hbm, ones_hbm, q_hbm, counts_hbm,
                  idx_v, rows_v, ones_v, cshared, sem)


def _perp_kernel(counts_ref, perp_ref):
    counts = counts_ref[0, :] + counts_ref[1, :]
    avg = counts / jnp.float32(_K)
    perp_ref[...] = jnp.exp(
        -jnp.sum(avg * jnp.log(avg + 1e-10))).reshape(1, 1)


def _tc_perplexity(counts2):
    return pl.pallas_call(
        _perp_kernel,
        out_shape=jax.ShapeDtypeStruct((1, 1), jnp.float32),
    )(counts2)


def kernel(z, embedding, cluster_size):
    B, C, D, H, W = z.shape
    K = embedding.shape[0]
    n_tokens = B * D * H * W

    flat_xt = jnp.transpose(z, (1, 0, 2, 3, 4)).reshape(C, n_tokens)
    cs2d = cluster_size.reshape(1, K)

    idx3, loss, used = _tc_argmin(flat_xt, embedding, cs2d, n_tokens)
    idx_rows = idx3.reshape(n_tokens // 128, 128)

    emb_pad = jnp.pad(embedding, ((0, 0), (0, 128 - C)))
    zeros = jnp.zeros((K,), jnp.float32)
    ones = jnp.ones((128,), jnp.float32)
    qpad, counts2 = _sc_gather_hist(emb_pad, idx_rows, zeros, ones)

    perp = _tc_perplexity(counts2)

    q = qpad.reshape(n_tokens, 128)[:, :C]
    quantized = q.reshape(B, D, H, W, C).transpose(0, 4, 1, 2, 3)
    encoding_indices = idx3.reshape(B, D, H, W)
    return (quantized, loss[0, 0], encoding_indices, perp[0, 0], used[0, 0])
